# baseline (speedup 1.0000x reference)
import jax
import jax.numpy as jnp
from jax.experimental import pallas as pl

S = 2048
H = 768
HD = 4002
HOUT = 4000
D1, V1 = 192, 16000
D2, V2 = 48, 80000
V = HOUT + V1 + V2

RB = 128
CS = 3200

BW = 512
NB = (V + BW - 1) // BW
R2 = 2048
N1B = (V1 + BW - 1) // BW
N2B = (V2 + BW - 1) // BW
A1Q = HOUT // BW
A2Q = (HOUT + V1) // BW
NHB = (HD + BW - 1) // BW
SH1 = BW - HOUT % BW
SH2 = BW - (HOUT + V1) % BW

NEG = -1e30


def _stats_kernel(x_ref, hwt_ref, hbt_ref, p1wt_ref, p1bt_ref, t1w_ref,
                  t1b_ref, p2wt_ref, p2bt_ref, t2w_ref, t2b_ref,
                  headt_ref, proj1t_ref, proj2t_ref, statst_ref):
    xt = x_ref[0].astype(jnp.bfloat16).T
    zh = jnp.dot(hwt_ref[:].astype(jnp.bfloat16), xt,
                 preferred_element_type=jnp.float32)
    zh = zh + hbt_ref[:, 0][:, None]
    mh = jnp.max(zh, axis=0, keepdims=True)
    eh = jnp.exp(zh - mh)
    inv_sh = 1.0 / jnp.sum(eh, axis=0, keepdims=True)
    et = eh * inv_sh
    headt_ref[:] = et
    g1 = et[HOUT:HOUT + 1, :]
    g2 = et[HOUT + 1:HOUT + 2, :]

    p1 = jnp.dot(p1wt_ref[:].astype(jnp.bfloat16), xt,
                 preferred_element_type=jnp.float32)
    p1 = (p1 + p1bt_ref[:, 0][:, None]).astype(jnp.bfloat16)
    proj1t_ref[:] = p1
    p2 = jnp.dot(p2wt_ref[:].astype(jnp.bfloat16), xt,
                 preferred_element_type=jnp.float32)
    p2 = (p2 + p2bt_ref[:, 0][:, None]).astype(jnp.bfloat16)
    proj2t_ref[:] = p2

    def tail_stats(p, w_ref, b_ref, v):
        def body(i, carry):
            m, s = carry
            sl = pl.ds(i * CS, CS)
            z = jnp.dot(p, w_ref[:, sl].astype(jnp.bfloat16),
                        preferred_element_type=jnp.float32)
            z = z + b_ref[0, sl][None, :]
            mc = jnp.max(z, axis=-1, keepdims=True)
            mn = jnp.maximum(m, mc)
            s = s * jnp.exp(m - mn) + jnp.sum(jnp.exp(z - mn), axis=-1,
                                              keepdims=True)
            return mn, s
        m0 = jnp.full((p.shape[0], 1), NEG, dtype=jnp.float32)
        s0 = jnp.zeros((p.shape[0], 1), dtype=jnp.float32)
        return jax.lax.fori_loop(0, v // CS, body, (m0, s0))

    m1, s1 = tail_stats(p1.T, t1w_ref, t1b_ref, V1)
    m2, s2 = tail_stats(p2.T, t2w_ref, t2b_ref, V2)

    zeros = jnp.zeros_like(m1)
    statst_ref[:] = jnp.concatenate(
        [m1, g1.T / s1, m2, g2.T / s2, zeros, zeros, zeros, zeros], axis=1).T


def _write_kernel(headt_ref, proj1t_ref, proj2t_ref, statst_ref,
                  w1p_ref, w1c_ref, b1p_ref, b1c_ref,
                  w2p_ref, w2c_ref, b2p_ref, b2c_ref, out_ref):
    j = pl.program_id(1)

    def store(v):
        out_ref[...] = v[None, :, :]

    def t1_val():
        w = jnp.concatenate([w1p_ref[SH1:, :].astype(jnp.bfloat16),
                             w1c_ref[:SH1, :].astype(jnp.bfloat16)], axis=0)
        b = jnp.concatenate([b1p_ref[SH1:, :], b1c_ref[:SH1, :]], axis=0)
        z = jnp.dot(w, proj1t_ref[:], preferred_element_type=jnp.float32) + b
        return jnp.exp(z - statst_ref[0:1, :]) * statst_ref[1:2, :]

    def t2_val():
        w = jnp.concatenate([w2p_ref[SH2:, :].astype(jnp.bfloat16),
                             w2c_ref[:SH2, :].astype(jnp.bfloat16)], axis=0)
        b = jnp.concatenate([b2p_ref[SH2:, :], b2c_ref[:SH2, :]], axis=0)
        z = jnp.dot(w, proj2t_ref[:], preferred_element_type=jnp.float32) + b
        return jnp.exp(z - statst_ref[2:3, :]) * statst_ref[3:4, :]

    def rows():
        return (j * BW
                + jax.lax.broadcasted_iota(jnp.int32, (BW, 1), 0))

    @pl.when(j < A1Q)
    def _():
        store(headt_ref[:])

    @pl.when(j == A1Q)
    def _():
        store(jnp.where(rows() < HOUT, headt_ref[:], t1_val()))

    @pl.when(jnp.logical_and(j > A1Q, j < A2Q))
    def _():
        store(t1_val())

    @pl.when(j == A2Q)
    def _():
        store(jnp.where(rows() < HOUT + V1, t1_val(), t2_val()))

    @pl.when(j > A2Q)
    def _():
        store(t2_val())


def kernel(inp, head_w, head_b, t1_pw, t1_pb, t1_w, t1_b,
           t2_pw, t2_pb, t2_w, t2_b):
    hwt = head_w.T
    p1wt = t1_pw.T
    p2wt = t2_pw.T
    t1wt = t1_w.T
    t2wt = t2_w.T
    hbt = head_b.reshape(HD, 1)
    p1bt = t1_pb.reshape(D1, 1)
    p2bt = t2_pb.reshape(D2, 1)
    t1bt = t1_b.reshape(V1, 1)
    t2bt = t2_b.reshape(V2, 1)

    full = lambda shape: pl.BlockSpec(shape, lambda i: (0,) * len(shape))
    headt, proj1t, proj2t, statst = pl.pallas_call(
        _stats_kernel,
        grid=(S // RB,),
        in_specs=[
            pl.BlockSpec((1, RB, H), lambda i: (0, i, 0)),
            full((HD, H)), full((HD, 1)),
            full((D1, H)), full((D1, 1)), full((D1, V1)), full((1, V1)),
            full((D2, H)), full((D2, 1)), full((D2, V2)), full((1, V2)),
        ],
        out_specs=[
            pl.BlockSpec((HD, RB), lambda i: (0, i)),
            pl.BlockSpec((D1, RB), lambda i: (0, i)),
            pl.BlockSpec((D2, RB), lambda i: (0, i)),
            pl.BlockSpec((8, RB), lambda i: (0, i)),
        ],
        out_shape=[
            jax.ShapeDtypeStruct((HD, S), jnp.float32),
            jax.ShapeDtypeStruct((D1, S), jnp.bfloat16),
            jax.ShapeDtypeStruct((D2, S), jnp.bfloat16),
            jax.ShapeDtypeStruct((8, S), jnp.float32),
        ],
    )(inp, hwt, hbt, p1wt, p1bt, t1_w, t1_b.reshape(1, V1),
      p2wt, p2bt, t2_w, t2_b.reshape(1, V2))

    out = pl.pallas_call(
        _write_kernel,
        grid=(S // R2, NB),
        in_specs=[
            pl.BlockSpec((BW, R2), lambda i, j: (jnp.minimum(j, NHB - 1), i)),
            pl.BlockSpec((D1, R2), lambda i, j: (0, i)),
            pl.BlockSpec((D2, R2), lambda i, j: (0, i)),
            pl.BlockSpec((8, R2), lambda i, j: (0, i)),
            pl.BlockSpec((BW, D1), lambda i, j: (jnp.clip(j - A1Q - 1, 0, N1B - 1), 0)),
            pl.BlockSpec((BW, D1), lambda i, j: (jnp.clip(j - A1Q, 0, N1B - 1), 0)),
            pl.BlockSpec((BW, 1), lambda i, j: (jnp.clip(j - A1Q - 1, 0, N1B - 1), 0)),
            pl.BlockSpec((BW, 1), lambda i, j: (jnp.clip(j - A1Q, 0, N1B - 1), 0)),
            pl.BlockSpec((BW, D2), lambda i, j: (jnp.clip(j - A2Q - 1, 0, N2B - 1), 0)),
            pl.BlockSpec((BW, D2), lambda i, j: (jnp.clip(j - A2Q, 0, N2B - 1), 0)),
            pl.BlockSpec((BW, 1), lambda i, j: (jnp.clip(j - A2Q - 1, 0, N2B - 1), 0)),
            pl.BlockSpec((BW, 1), lambda i, j: (jnp.clip(j - A2Q, 0, N2B - 1), 0)),
        ],
        out_specs=pl.BlockSpec((1, BW, R2), lambda i, j: (0, j, i)),
        out_shape=jax.ShapeDtypeStruct((1, V, S), jnp.float32),
    )(headt, proj1t, proj2t, statst,
      t1wt, t1wt, t1bt, t1bt, t2wt, t2wt, t2bt, t2bt)

    return jnp.swapaxes(out, 1, 2)


# speedup vs baseline: 1.1022x; 1.1022x over previous
import jax
import jax.numpy as jnp
from jax.experimental import pallas as pl

S = 2048
H = 768
HD = 4002
HOUT = 4000
D1, V1 = 192, 16000
D2, V2 = 48, 80000
V = HOUT + V1 + V2

RB = 256
CS = 3200

BW = 512
NB = (V + BW - 1) // BW
R2 = 2048
N1B = (V1 + BW - 1) // BW
N2B = (V2 + BW - 1) // BW
A1Q = HOUT // BW
A2Q = (HOUT + V1) // BW
NHB = (HD + BW - 1) // BW
SH1 = BW - HOUT % BW
SH2 = BW - (HOUT + V1) % BW

NEG = -1e30


def _stats_kernel(x_ref, hwt_ref, hbt_ref, p1wt_ref, p1bt_ref, t1w_ref,
                  t1b_ref, p2wt_ref, p2bt_ref, t2w_ref, t2b_ref,
                  headt_ref, proj1t_ref, proj2t_ref, statst_ref):
    xt = x_ref[0].astype(jnp.bfloat16).T
    zh = jnp.dot(hwt_ref[:].astype(jnp.bfloat16), xt,
                 preferred_element_type=jnp.float32)
    zh = zh + hbt_ref[:, 0][:, None]
    mh = jnp.max(zh, axis=0, keepdims=True)
    eh = jnp.exp(zh - mh)
    inv_sh = 1.0 / jnp.sum(eh, axis=0, keepdims=True)
    et = eh * inv_sh
    headt_ref[:] = et
    g1 = et[HOUT:HOUT + 1, :]
    g2 = et[HOUT + 1:HOUT + 2, :]

    p1 = jnp.dot(p1wt_ref[:].astype(jnp.bfloat16), xt,
                 preferred_element_type=jnp.float32)
    p1 = (p1 + p1bt_ref[:, 0][:, None]).astype(jnp.bfloat16)
    proj1t_ref[:] = p1
    p2 = jnp.dot(p2wt_ref[:].astype(jnp.bfloat16), xt,
                 preferred_element_type=jnp.float32)
    p2 = (p2 + p2bt_ref[:, 0][:, None]).astype(jnp.bfloat16)
    proj2t_ref[:] = p2

    def tail_stats(p, w_ref, b_ref, v):
        def body(i, carry):
            m, s = carry
            sl = pl.ds(i * CS, CS)
            z = jnp.dot(p, w_ref[:, sl].astype(jnp.bfloat16),
                        preferred_element_type=jnp.float32)
            z = z + b_ref[0, sl][None, :]
            mc = jnp.max(z, axis=-1, keepdims=True)
            mn = jnp.maximum(m, mc)
            s = s * jnp.exp(m - mn) + jnp.sum(jnp.exp(z - mn), axis=-1,
                                              keepdims=True)
            return mn, s
        m0 = jnp.full((p.shape[0], 1), NEG, dtype=jnp.float32)
        s0 = jnp.zeros((p.shape[0], 1), dtype=jnp.float32)
        return jax.lax.fori_loop(0, v // CS, body, (m0, s0))

    m1, s1 = tail_stats(p1.T, t1w_ref, t1b_ref, V1)
    m2, s2 = tail_stats(p2.T, t2w_ref, t2b_ref, V2)

    zeros = jnp.zeros_like(m1)
    statst_ref[:] = jnp.concatenate(
        [m1, g1.T / s1, m2, g2.T / s2, zeros, zeros, zeros, zeros], axis=1).T


def _write_kernel(headt_ref, proj1t_ref, proj2t_ref, statst_ref,
                  w1p_ref, w1c_ref, b1p_ref, b1c_ref,
                  w2p_ref, w2c_ref, b2p_ref, b2c_ref, out_ref):
    j = pl.program_id(1)

    def store(v):
        out_ref[...] = v[None, :, :]

    def t1_val():
        w = jnp.concatenate([w1p_ref[SH1:, :].astype(jnp.bfloat16),
                             w1c_ref[:SH1, :].astype(jnp.bfloat16)], axis=0)
        b = jnp.concatenate([b1p_ref[SH1:, :], b1c_ref[:SH1, :]], axis=0)
        z = jnp.dot(w, proj1t_ref[:], preferred_element_type=jnp.float32) + b
        return jnp.exp(z - statst_ref[0:1, :]) * statst_ref[1:2, :]

    def t2_val():
        w = jnp.concatenate([w2p_ref[SH2:, :].astype(jnp.bfloat16),
                             w2c_ref[:SH2, :].astype(jnp.bfloat16)], axis=0)
        b = jnp.concatenate([b2p_ref[SH2:, :], b2c_ref[:SH2, :]], axis=0)
        z = jnp.dot(w, proj2t_ref[:], preferred_element_type=jnp.float32) + b
        return jnp.exp(z - statst_ref[2:3, :]) * statst_ref[3:4, :]

    def rows():
        return (j * BW
                + jax.lax.broadcasted_iota(jnp.int32, (BW, 1), 0))

    @pl.when(j < A1Q)
    def _():
        store(headt_ref[:])

    @pl.when(j == A1Q)
    def _():
        store(jnp.where(rows() < HOUT, headt_ref[:], t1_val()))

    @pl.when(jnp.logical_and(j > A1Q, j < A2Q))
    def _():
        store(t1_val())

    @pl.when(j == A2Q)
    def _():
        store(jnp.where(rows() < HOUT + V1, t1_val(), t2_val()))

    @pl.when(j > A2Q)
    def _():
        store(t2_val())


def kernel(inp, head_w, head_b, t1_pw, t1_pb, t1_w, t1_b,
           t2_pw, t2_pb, t2_w, t2_b):
    hwt = head_w.T
    p1wt = t1_pw.T
    p2wt = t2_pw.T
    t1wt = t1_w.T
    t2wt = t2_w.T
    hbt = head_b.reshape(HD, 1)
    p1bt = t1_pb.reshape(D1, 1)
    p2bt = t2_pb.reshape(D2, 1)
    t1bt = t1_b.reshape(V1, 1)
    t2bt = t2_b.reshape(V2, 1)

    full = lambda shape: pl.BlockSpec(shape, lambda i: (0,) * len(shape))
    headt, proj1t, proj2t, statst = pl.pallas_call(
        _stats_kernel,
        grid=(S // RB,),
        in_specs=[
            pl.BlockSpec((1, RB, H), lambda i: (0, i, 0)),
            full((HD, H)), full((HD, 1)),
            full((D1, H)), full((D1, 1)), full((D1, V1)), full((1, V1)),
            full((D2, H)), full((D2, 1)), full((D2, V2)), full((1, V2)),
        ],
        out_specs=[
            pl.BlockSpec((HD, RB), lambda i: (0, i)),
            pl.BlockSpec((D1, RB), lambda i: (0, i)),
            pl.BlockSpec((D2, RB), lambda i: (0, i)),
            pl.BlockSpec((8, RB), lambda i: (0, i)),
        ],
        out_shape=[
            jax.ShapeDtypeStruct((HD, S), jnp.float32),
            jax.ShapeDtypeStruct((D1, S), jnp.bfloat16),
            jax.ShapeDtypeStruct((D2, S), jnp.bfloat16),
            jax.ShapeDtypeStruct((8, S), jnp.float32),
        ],
    )(inp, hwt, hbt, p1wt, p1bt, t1_w, t1_b.reshape(1, V1),
      p2wt, p2bt, t2_w, t2_b.reshape(1, V2))

    out = pl.pallas_call(
        _write_kernel,
        grid=(S // R2, NB),
        in_specs=[
            pl.BlockSpec((BW, R2), lambda i, j: (jnp.minimum(j, NHB - 1), i)),
            pl.BlockSpec((D1, R2), lambda i, j: (0, i)),
            pl.BlockSpec((D2, R2), lambda i, j: (0, i)),
            pl.BlockSpec((8, R2), lambda i, j: (0, i)),
            pl.BlockSpec((BW, D1), lambda i, j: (jnp.clip(j - A1Q - 1, 0, N1B - 1), 0)),
            pl.BlockSpec((BW, D1), lambda i, j: (jnp.clip(j - A1Q, 0, N1B - 1), 0)),
            pl.BlockSpec((BW, 1), lambda i, j: (jnp.clip(j - A1Q - 1, 0, N1B - 1), 0)),
            pl.BlockSpec((BW, 1), lambda i, j: (jnp.clip(j - A1Q, 0, N1B - 1), 0)),
            pl.BlockSpec((BW, D2), lambda i, j: (jnp.clip(j - A2Q - 1, 0, N2B - 1), 0)),
            pl.BlockSpec((BW, D2), lambda i, j: (jnp.clip(j - A2Q, 0, N2B - 1), 0)),
            pl.BlockSpec((BW, 1), lambda i, j: (jnp.clip(j - A2Q - 1, 0, N2B - 1), 0)),
            pl.BlockSpec((BW, 1), lambda i, j: (jnp.clip(j - A2Q, 0, N2B - 1), 0)),
        ],
        out_specs=pl.BlockSpec((1, BW, R2), lambda i, j: (0, j, i)),
        out_shape=jax.ShapeDtypeStruct((1, V, S), jnp.float32),
    )(headt, proj1t, proj2t, statst,
      t1wt, t1wt, t1bt, t1bt, t2wt, t2wt, t2bt, t2bt)

    return jnp.swapaxes(out, 1, 2)
